# Initial kernel scaffold; baseline (speedup 1.0000x reference)
import jax
import jax.numpy as jnp
from jax.experimental import pallas as pl


def kernel(coords, feats, W_in, b_in, g1_kw1, g1_kb1, g1_kw2, g1_kb2, g1_vw, g1_vb, g1_lng, g1_lnb, g1_sew1, g1_sew2, g2_kw1, g2_kb1, g2_kw2, g2_kb2, g2_vw, g2_vb, g2_lng, g2_lnb, g2_sew1, g2_sew2, W_lat, b_lat, p_kw1, p_kb1, p_kw2, p_kb2, p_lng, p_lnb):
    raise NotImplementedError("write your pallas kernel here")


# speedup vs baseline: 10.8147x; 10.8147x over previous
import functools

import numpy as np
import jax
import jax.numpy as jnp
from jax import lax
from jax.experimental import pallas as pl
from jax.experimental.pallas import tpu as pltpu
from jax.experimental.pallas import tpu_sc as plsc

N = 8192
KG = 12
KP = 8
GM = 32 * 32 * 32
PAD1 = 128
PAD2 = 256
BIGI = np.int32(2 ** 30)


def _dot(a, b):
    return lax.dot_general(
        a.astype(jnp.bfloat16), b.astype(jnp.bfloat16),
        (((1,), (0,)), ((), ())), preferred_element_type=jnp.float32)


def _gelu(x):
    return x * (lax.erf(x / np.sqrt(2).astype(np.float32)) + 1.0) / 2.0


def _sigmoid(x):
    return 1.0 / (1.0 + jnp.exp(-x))


def _ln(x, g, b, eps=1e-5):
    m = jnp.mean(x, axis=-1, keepdims=True)
    v = jnp.mean((x - m) * (x - m), axis=-1, keepdims=True)
    return (x - m) / jnp.sqrt(v + eps) * g + b


BRK = 256


def _knn_body(ct_ref, a_ref, dst_ref):
    ct = ct_ref[...]
    a = a_ref[...]
    b2 = jnp.sum(ct * ct, axis=0, keepdims=True)
    a2 = jnp.sum(a * a, axis=1, keepdims=True)
    d2 = jnp.maximum(a2 + b2 - 2.0 * _dot(a, ct), 0.0)
    i = pl.program_id(0)
    rows = i * BRK + lax.broadcasted_iota(jnp.int32, (BRK, N), 0)
    cols = lax.broadcasted_iota(jnp.int32, (BRK, N), 1)
    d2 = jnp.where(cols == rows, jnp.inf, d2)
    outs = []
    for _ in range(KG):
        m = jnp.min(d2, axis=1, keepdims=True)
        j = jnp.min(jnp.where(d2 == m, cols, BIGI), axis=1, keepdims=True)
        outs.append(j)
        d2 = jnp.where(cols == j, jnp.inf, d2)
    dst_ref[...] = jnp.concatenate(outs, axis=1)


def _knn(ct, c2):
    return pl.pallas_call(
        _knn_body,
        grid=(N // BRK,),
        in_specs=[pl.BlockSpec((3, N), lambda i: (0, 0)),
                  pl.BlockSpec((BRK, 3), lambda i: (i, 0))],
        out_specs=pl.BlockSpec((BRK, KG), lambda i: (i, 0)),
        out_shape=jax.ShapeDtypeStruct((N, KG), jnp.int32),
    )(ct, c2)


BRG = 256


def _gknn_body(ct_ref, g_ref, dist_ref, idx_ref):
    ct = ct_ref[...]
    a = g_ref[...]
    b2 = jnp.sum(ct * ct, axis=0, keepdims=True)
    a2 = jnp.sum(a * a, axis=1, keepdims=True)
    d2 = jnp.maximum(a2 + b2 - 2.0 * _dot(a, ct), 0.0)
    cols = lax.broadcasted_iota(jnp.int32, (BRG, N), 1)
    dcols, icols = [], []
    for _ in range(KP):
        m = jnp.min(d2, axis=1, keepdims=True)
        j = jnp.min(jnp.where(d2 == m, cols, BIGI), axis=1, keepdims=True)
        dcols.append(jnp.sqrt(m))
        icols.append(j)
        d2 = jnp.where(cols == j, jnp.inf, d2)
    dist_ref[...] = jnp.concatenate(dcols, axis=1)
    idx_ref[...] = jnp.concatenate(icols, axis=1)


def _gknn(ct, grid_c):
    return pl.pallas_call(
        _gknn_body,
        grid=(GM // BRG,),
        in_specs=[pl.BlockSpec((3, N), lambda i: (0, 0)),
                  pl.BlockSpec((BRG, 3), lambda i: (i, 0))],
        out_specs=[pl.BlockSpec((BRG, KP), lambda i: (i, 0)),
                   pl.BlockSpec((BRG, KP), lambda i: (i, 0))],
        out_shape=[jax.ShapeDtypeStruct((GM, KP), jnp.float32),
                   jax.ShapeDtypeStruct((GM, KP), jnp.int32)],
    )(ct, grid_c)


def _tbl_in_body(f_ref, c_ref, win_ref, bin_ref, vw_ref, vb_ref, tbl_ref):
    x = _dot(f_ref[...], win_ref[...]) + bin_ref[...]
    v = _dot(x, vw_ref[...]) + vb_ref[...]
    z = jnp.zeros((N, PAD1 - 67), jnp.float32)
    tbl_ref[...] = jnp.concatenate([v, c_ref[...], z], axis=1)


def _tbl_in(f2, c2, w_in, b_in, vw, vb):
    return pl.pallas_call(
        _tbl_in_body,
        out_shape=jax.ShapeDtypeStruct((N, PAD1), jnp.float32),
    )(f2, c2, w_in, b_in, vw, vb)


BRN = 512
NBN = N // BRN


def _gno_body(ga_ref, tbl_ref, kw1_ref, kb1_ref, kw2_ref, kb2_ref,
              out_ref, bs_ref):
    tbl = tbl_ref[...]
    c = tbl[:, 64:67]
    kw1 = kw1_ref[...]
    kb1 = kb1_ref[...]
    kw2 = kw2_ref[...]
    kb2 = kb2_ref[...]
    ga = ga_ref[...]
    acc = jnp.zeros((BRN, 64), jnp.float32)
    for k in range(KG):
        g = ga[:, k * PAD1:(k + 1) * PAD1]
        rel = g[:, 64:67] - c
        kap = _dot(_gelu(_dot(rel, kw1) + kb1), kw2) + kb2
        acc = acc + kap * g[:, 0:64]
    out_ref[...] = acc
    bs_ref[...] = jnp.sum(acc, axis=0)[None, None, :]


def _gno(ga, tbl, kw1, kb1, kw2, kb2):
    return pl.pallas_call(
        _gno_body,
        grid=(NBN,),
        in_specs=[pl.BlockSpec((BRN, KG * PAD1), lambda i: (i, 0)),
                  pl.BlockSpec((BRN, PAD1), lambda i: (i, 0)),
                  pl.BlockSpec((3, 32), lambda i: (0, 0)),
                  pl.BlockSpec((1, 32), lambda i: (0, 0)),
                  pl.BlockSpec((32, 64), lambda i: (0, 0)),
                  pl.BlockSpec((1, 64), lambda i: (0, 0))],
        out_specs=[pl.BlockSpec((BRN, 64), lambda i: (i, 0)),
                   pl.BlockSpec((1, 1, 64), lambda i: (i, 0, 0))],
        out_shape=[jax.ShapeDtypeStruct((N, 64), jnp.float32),
                   jax.ShapeDtypeStruct((NBN, 1, 64), jnp.float32)],
    )(ga, tbl, kw1, kb1, kw2, kb2)


def _scale_body(out_ref, tbl_ref, bs_ref, sew1_ref, sew2_ref, lng_ref,
                lnb_ref, pw_ref, pb_ref, ntbl_ref, *, pad):
    ym = jnp.sum(bs_ref[...], axis=0, keepdims=True) / np.float32(N)
    sc = _sigmoid(_dot(_gelu(_dot(ym, sew1_ref[...])), sew2_ref[...]))
    tbl = tbl_ref[...]
    v = tbl[:, 0:64]
    c = tbl[:, 64:67]
    x2 = _ln(_gelu(out_ref[...] * sc + v), lng_ref[...], lnb_ref[...])
    nv = _dot(x2, pw_ref[...]) + pb_ref[...]
    z = jnp.zeros((BRN, pad - nv.shape[1] - 3), jnp.float32)
    ntbl_ref[...] = jnp.concatenate([nv, c, z], axis=1)


def _scale(out, tbl, bs, sew1, sew2, lng, lnb, pw, pb, pad):
    pd = pw.shape[1]
    return pl.pallas_call(
        functools.partial(_scale_body, pad=pad),
        grid=(NBN,),
        in_specs=[pl.BlockSpec((BRN, 64), lambda i: (i, 0)),
                  pl.BlockSpec((BRN, PAD1), lambda i: (i, 0)),
                  pl.BlockSpec((NBN, 64), lambda i: (0, 0)),
                  pl.BlockSpec((64, 16), lambda i: (0, 0)),
                  pl.BlockSpec((16, 64), lambda i: (0, 0)),
                  pl.BlockSpec((1, 64), lambda i: (0, 0)),
                  pl.BlockSpec((1, 64), lambda i: (0, 0)),
                  pl.BlockSpec((64, pd), lambda i: (0, 0)),
                  pl.BlockSpec((1, pd), lambda i: (0, 0))],
        out_specs=pl.BlockSpec((BRN, pad), lambda i: (i, 0)),
        out_shape=jax.ShapeDtypeStruct((N, pad), jnp.float32),
    )(out, tbl, bs, sew1, sew2, lng, lnb, pw, pb)


BRP = 256


def _proj_body(ga_ref, dist_ref, gc_ref, kw1_ref, kb1_ref, kw2_ref,
               kb2_ref, lng_ref, lnb_ref, out_ref):
    w = 1.0 / (dist_ref[...] + 1e-6)
    w = w / jnp.sum(w, axis=1, keepdims=True)
    gc = gc_ref[...]
    kw1 = kw1_ref[...]
    kb1 = kb1_ref[...]
    kw2 = kw2_ref[...]
    kb2 = kb2_ref[...]
    ga = ga_ref[...]
    acc = jnp.zeros((BRP, 128), jnp.float32)
    for k in range(KP):
        g = ga[:, k * PAD2:(k + 1) * PAD2]
        rel = gc - g[:, 128:131]
        kap = _dot(_gelu(_dot(rel, kw1) + kb1), kw2) + kb2
        acc = acc + kap * g[:, 0:128] * w[:, k:k + 1]
    out_ref[...] = _ln(acc, lng_ref[...], lnb_ref[...])


def _proj(ga, dist8, grid_c, kw1, kb1, kw2, kb2, lng, lnb):
    return pl.pallas_call(
        _proj_body,
        grid=(GM // BRP,),
        in_specs=[pl.BlockSpec((BRP, KP * PAD2), lambda i: (i, 0)),
                  pl.BlockSpec((BRP, KP), lambda i: (i, 0)),
                  pl.BlockSpec((BRP, 3), lambda i: (i, 0)),
                  pl.BlockSpec((3, 128), lambda i: (0, 0)),
                  pl.BlockSpec((1, 128), lambda i: (0, 0)),
                  pl.BlockSpec((128, 128), lambda i: (0, 0)),
                  pl.BlockSpec((1, 128), lambda i: (0, 0)),
                  pl.BlockSpec((1, 128), lambda i: (0, 0)),
                  pl.BlockSpec((1, 128), lambda i: (0, 0))],
        out_specs=pl.BlockSpec((BRP, 128), lambda i: (i, 0)),
        out_shape=jax.ShapeDtypeStruct((GM, 128), jnp.float32),
    )(ga, dist8, grid_c, kw1, kb1, kw2, kb2, lng, lnb)


NW = 32


def _sc_gather(table, idx, D, B):
    SC_CH = 512 if D <= 128 else 256
    per = B // NW
    nch = per // SC_CH
    mesh = plsc.VectorSubcoreMesh(core_axis_name="c", subcore_axis_name="s")

    @functools.partial(
        pl.kernel, mesh=mesh,
        out_type=jax.ShapeDtypeStruct((B, D), jnp.float32),
        scratch_types=[pltpu.VMEM((SC_CH,), jnp.int32),
                       pltpu.VMEM((SC_CH, D), jnp.float32),
                       pltpu.SemaphoreType.DMA],
    )
    def k(table_hbm, idx_hbm, out_hbm, idx_v, rows_v, sem):
        wid = lax.axis_index("s") * 2 + lax.axis_index("c")
        base = wid * per

        def body(i, carry):
            b = base + i * SC_CH
            pltpu.sync_copy(idx_hbm.at[pl.ds(b, SC_CH)], idx_v)
            pltpu.async_copy(table_hbm.at[idx_v], rows_v, sem).wait()
            pltpu.sync_copy(rows_v, out_hbm.at[pl.ds(b, SC_CH)])
            return carry

        lax.fori_loop(0, nch, body, 0)

    return k(table, idx)


def kernel(coords, feats, W_in, b_in,
           g1_kw1, g1_kb1, g1_kw2, g1_kb2, g1_vw, g1_vb, g1_lng, g1_lnb,
           g1_sew1, g1_sew2,
           g2_kw1, g2_kb1, g2_kw2, g2_kb2, g2_vw, g2_vb, g2_lng, g2_lnb,
           g2_sew1, g2_sew2,
           W_lat, b_lat, p_kw1, p_kb1, p_kw2, p_kb2, p_lng, p_lnb):
    c2 = coords.reshape(N, 3)
    f2 = feats.reshape(N, 9)
    ct = c2.T

    dst = _knn(ct, c2)

    tbl1 = _tbl_in(f2, c2, W_in, b_in.reshape(1, -1),
                   g1_vw, g1_vb.reshape(1, -1))
    ga1 = _sc_gather(tbl1, dst.reshape(-1), PAD1, N * KG)
    o1, bs1 = _gno(ga1.reshape(N, KG * PAD1), tbl1,
                   g1_kw1, g1_kb1.reshape(1, -1), g1_kw2,
                   g1_kb2.reshape(1, -1))
    tbl2 = _scale(o1, tbl1, bs1.reshape(NBN, 64), g1_sew1, g1_sew2,
                  g1_lng.reshape(1, -1), g1_lnb.reshape(1, -1),
                  g2_vw, g2_vb.reshape(1, -1), PAD1)

    ga2 = _sc_gather(tbl2, dst.reshape(-1), PAD1, N * KG)
    o2, bs2 = _gno(ga2.reshape(N, KG * PAD1), tbl2,
                   g2_kw1, g2_kb1.reshape(1, -1), g2_kw2,
                   g2_kb2.reshape(1, -1))
    tbl3 = _scale(o2, tbl2, bs2.reshape(NBN, 64), g2_sew1, g2_sew2,
                  g2_lng.reshape(1, -1), g2_lnb.reshape(1, -1),
                  W_lat, b_lat.reshape(1, -1), PAD2)

    z = jnp.linspace(-1.0, 1.0, 32)
    gz, gy, gx = jnp.meshgrid(z, z, z, indexing='ij')
    grid_c = jnp.stack([gx, gy, gz], axis=-1).reshape(GM, 3)

    dist8, idx8 = _gknn(ct, grid_c)
    ga3 = _sc_gather(tbl3, idx8.reshape(-1), PAD2, GM * KP)
    out = _proj(ga3.reshape(GM, KP * PAD2), dist8, grid_c,
                p_kw1, p_kb1.reshape(1, -1), p_kw2, p_kb2.reshape(1, -1),
                p_lng.reshape(1, -1), p_lnb.reshape(1, -1))

    return out.reshape(1, 32, 32, 32, 128).transpose(0, 4, 1, 2, 3)


# speedup vs baseline: 11.8565x; 1.0963x over previous
import functools

import numpy as np
import jax
import jax.numpy as jnp
from jax import lax
from jax.experimental import pallas as pl
from jax.experimental.pallas import tpu as pltpu
from jax.experimental.pallas import tpu_sc as plsc

N = 8192
KG = 12
KP = 8
GM = 32 * 32 * 32
PAD1 = 128
PAD2 = 256
BIGI = np.int32(2 ** 30)


def _dot(a, b):
    return lax.dot_general(
        a.astype(jnp.bfloat16), b.astype(jnp.bfloat16),
        (((1,), (0,)), ((), ())), preferred_element_type=jnp.float32)


def _gelu(x):
    return x * (lax.erf(x / np.sqrt(2).astype(np.float32)) + 1.0) / 2.0


def _sigmoid(x):
    return 1.0 / (1.0 + jnp.exp(-x))


def _ln(x, g, b, eps=1e-5):
    m = jnp.mean(x, axis=-1, keepdims=True)
    v = jnp.mean((x - m) * (x - m), axis=-1, keepdims=True)
    return (x - m) / jnp.sqrt(v + eps) * g + b


BRK = 256


IDXM = np.int32(8191)
IMAX = np.int32(2 ** 31 - 1)


def _pack(d2, cols):
    d2i = lax.bitcast_convert_type(d2, jnp.int32)
    return (d2i & ~IDXM) | cols


def _knn_body(ct_ref, a_ref, dst_ref):
    ct = ct_ref[...]
    a = a_ref[...]
    b2 = jnp.sum(ct * ct, axis=0, keepdims=True)
    a2 = jnp.sum(a * a, axis=1, keepdims=True)
    d2 = jnp.maximum(a2 + b2 - 2.0 * _dot(a, ct), 0.0)
    i = pl.program_id(0)
    rows = i * BRK + lax.broadcasted_iota(jnp.int32, (BRK, N), 0)
    cols = lax.broadcasted_iota(jnp.int32, (BRK, N), 1)
    d2 = jnp.where(cols == rows, jnp.inf, d2)
    outs = []
    for _ in range(KG):
        m = jnp.min(d2, axis=1, keepdims=True)
        j = jnp.min(jnp.where(d2 == m, cols, BIGI), axis=1, keepdims=True)
        outs.append(j)
        d2 = jnp.where(cols == j, jnp.inf, d2)
    dst_ref[...] = jnp.concatenate(outs, axis=1)


def _knn(ct, c2):
    return pl.pallas_call(
        _knn_body,
        grid=(N // BRK,),
        in_specs=[pl.BlockSpec((3, N), lambda i: (0, 0)),
                  pl.BlockSpec((BRK, 3), lambda i: (i, 0))],
        out_specs=pl.BlockSpec((BRK, KG), lambda i: (i, 0)),
        out_shape=jax.ShapeDtypeStruct((N, KG), jnp.int32),
    )(ct, c2)


BRG = 256


def _gknn_body(ct_ref, g_ref, dist_ref, idx_ref):
    ct = ct_ref[...]
    a = g_ref[...]
    b2 = jnp.sum(ct * ct, axis=0, keepdims=True)
    a2 = jnp.sum(a * a, axis=1, keepdims=True)
    d2 = jnp.maximum(a2 + b2 - 2.0 * _dot(a, ct), 0.0)
    cols = lax.broadcasted_iota(jnp.int32, (BRG, N), 1)
    dcols, icols = [], []
    for _ in range(KP):
        m = jnp.min(d2, axis=1, keepdims=True)
        j = jnp.min(jnp.where(d2 == m, cols, BIGI), axis=1, keepdims=True)
        icols.append(j)
        dcols.append(m)
        d2 = jnp.where(cols == j, jnp.inf, d2)
    dist_ref[...] = jnp.sqrt(jnp.concatenate(dcols, axis=1))
    idx_ref[...] = jnp.concatenate(icols, axis=1)


def _gknn(ct, grid_c):
    return pl.pallas_call(
        _gknn_body,
        grid=(GM // BRG,),
        in_specs=[pl.BlockSpec((3, N), lambda i: (0, 0)),
                  pl.BlockSpec((BRG, 3), lambda i: (i, 0))],
        out_specs=[pl.BlockSpec((BRG, KP), lambda i: (i, 0)),
                   pl.BlockSpec((BRG, KP), lambda i: (i, 0))],
        out_shape=[jax.ShapeDtypeStruct((GM, KP), jnp.float32),
                   jax.ShapeDtypeStruct((GM, KP), jnp.int32)],
    )(ct, grid_c)


def _tbl_in_body(f_ref, c_ref, win_ref, bin_ref, vw_ref, vb_ref, tbl_ref):
    x = _dot(f_ref[...], win_ref[...]) + bin_ref[...]
    v = _dot(x, vw_ref[...]) + vb_ref[...]
    z = jnp.zeros((N, PAD1 - 67), jnp.float32)
    tbl_ref[...] = jnp.concatenate([v, c_ref[...], z], axis=1)


def _tbl_in(f2, c2, w_in, b_in, vw, vb):
    return pl.pallas_call(
        _tbl_in_body,
        out_shape=jax.ShapeDtypeStruct((N, PAD1), jnp.float32),
    )(f2, c2, w_in, b_in, vw, vb)


BRN = 512
NBN = N // BRN


BRN2 = 256
NBG = N // BRN2


def _gno_body(ga_ref, tbl_ref, kw1_ref, kb1_ref, kw2_ref, kb2_ref,
              out_ref, bs_ref):
    tbl = tbl_ref[...]
    c = tbl[:, 64:67]
    ga = ga_ref[...].reshape(BRN2, KG, PAD1)
    rel = (ga[:, :, 64:67] - c[:, None, :]).reshape(BRN2 * KG, 3)
    kap = _dot(_gelu(_dot(rel, kw1_ref[...]) + kb1_ref[...]),
               kw2_ref[...]) + kb2_ref[...]
    msg = kap.reshape(BRN2, KG, 64) * ga[:, :, 0:64]
    acc = jnp.sum(msg, axis=1)
    out_ref[...] = acc
    bs_ref[...] = jnp.sum(acc, axis=0)[None, None, :]


def _gno(ga, tbl, kw1, kb1, kw2, kb2):
    return pl.pallas_call(
        _gno_body,
        grid=(NBG,),
        in_specs=[pl.BlockSpec((BRN2 * KG, PAD1), lambda i: (i, 0)),
                  pl.BlockSpec((BRN2, PAD1), lambda i: (i, 0)),
                  pl.BlockSpec((3, 32), lambda i: (0, 0)),
                  pl.BlockSpec((1, 32), lambda i: (0, 0)),
                  pl.BlockSpec((32, 64), lambda i: (0, 0)),
                  pl.BlockSpec((1, 64), lambda i: (0, 0))],
        out_specs=[pl.BlockSpec((BRN2, 64), lambda i: (i, 0)),
                   pl.BlockSpec((1, 1, 64), lambda i: (i, 0, 0))],
        out_shape=[jax.ShapeDtypeStruct((N, 64), jnp.float32),
                   jax.ShapeDtypeStruct((NBG, 1, 64), jnp.float32)],
    )(ga, tbl, kw1, kb1, kw2, kb2)


def _scale_body(out_ref, tbl_ref, bs_ref, sew1_ref, sew2_ref, lng_ref,
                lnb_ref, pw_ref, pb_ref, ntbl_ref, *, pad):
    ym = jnp.sum(bs_ref[...], axis=0, keepdims=True) / np.float32(N)
    sc = _sigmoid(_dot(_gelu(_dot(ym, sew1_ref[...])), sew2_ref[...]))
    tbl = tbl_ref[...]
    v = tbl[:, 0:64]
    c = tbl[:, 64:67]
    x2 = _ln(_gelu(out_ref[...] * sc + v), lng_ref[...], lnb_ref[...])
    nv = _dot(x2, pw_ref[...]) + pb_ref[...]
    z = jnp.zeros((BRN, pad - nv.shape[1] - 3), jnp.float32)
    ntbl_ref[...] = jnp.concatenate([nv, c, z], axis=1)


def _scale(out, tbl, bs, sew1, sew2, lng, lnb, pw, pb, pad):
    pd = pw.shape[1]
    return pl.pallas_call(
        functools.partial(_scale_body, pad=pad),
        grid=(NBN,),
        in_specs=[pl.BlockSpec((BRN, 64), lambda i: (i, 0)),
                  pl.BlockSpec((BRN, PAD1), lambda i: (i, 0)),
                  pl.BlockSpec((NBG, 64), lambda i: (0, 0)),
                  pl.BlockSpec((64, 16), lambda i: (0, 0)),
                  pl.BlockSpec((16, 64), lambda i: (0, 0)),
                  pl.BlockSpec((1, 64), lambda i: (0, 0)),
                  pl.BlockSpec((1, 64), lambda i: (0, 0)),
                  pl.BlockSpec((64, pd), lambda i: (0, 0)),
                  pl.BlockSpec((1, pd), lambda i: (0, 0))],
        out_specs=pl.BlockSpec((BRN, pad), lambda i: (i, 0)),
        out_shape=jax.ShapeDtypeStruct((N, pad), jnp.float32),
    )(out, tbl, bs, sew1, sew2, lng, lnb, pw, pb)


BRP = 256


def _proj_body(ga_ref, dist_ref, gc_ref, kw1_ref, kb1_ref, kw2_ref,
               kb2_ref, lng_ref, lnb_ref, out_ref):
    w = 1.0 / (dist_ref[...] + 1e-6)
    w = w / jnp.sum(w, axis=1, keepdims=True)
    gc = gc_ref[...]
    ga = ga_ref[...].reshape(BRP, KP, PAD2)
    rel = (gc[:, None, :] - ga[:, :, 128:131]).reshape(BRP * KP, 3)
    kap = _dot(_gelu(_dot(rel, kw1_ref[...]) + kb1_ref[...]),
               kw2_ref[...]) + kb2_ref[...]
    msg = kap.reshape(BRP, KP, 128) * ga[:, :, 0:128] * w[:, :, None]
    acc = jnp.sum(msg, axis=1)
    out_ref[...] = _ln(acc, lng_ref[...], lnb_ref[...])


def _proj(ga, dist8, grid_c, kw1, kb1, kw2, kb2, lng, lnb):
    return pl.pallas_call(
        _proj_body,
        grid=(GM // BRP,),
        in_specs=[pl.BlockSpec((BRP * KP, PAD2), lambda i: (i, 0)),
                  pl.BlockSpec((BRP, KP), lambda i: (i, 0)),
                  pl.BlockSpec((BRP, 3), lambda i: (i, 0)),
                  pl.BlockSpec((3, 128), lambda i: (0, 0)),
                  pl.BlockSpec((1, 128), lambda i: (0, 0)),
                  pl.BlockSpec((128, 128), lambda i: (0, 0)),
                  pl.BlockSpec((1, 128), lambda i: (0, 0)),
                  pl.BlockSpec((1, 128), lambda i: (0, 0)),
                  pl.BlockSpec((1, 128), lambda i: (0, 0))],
        out_specs=pl.BlockSpec((BRP, 128), lambda i: (i, 0)),
        out_shape=jax.ShapeDtypeStruct((GM, 128), jnp.float32),
    )(ga, dist8, grid_c, kw1, kb1, kw2, kb2, lng, lnb)


NW = 32


def _sc_gather(table, idx, D, B):
    SC_CH = 512 if D <= 128 else 256
    per = B // NW
    nch = per // SC_CH
    mesh = plsc.VectorSubcoreMesh(core_axis_name="c", subcore_axis_name="s")

    @functools.partial(
        pl.kernel, mesh=mesh,
        out_type=jax.ShapeDtypeStruct((B, D), jnp.float32),
        scratch_types=[pltpu.VMEM((SC_CH,), jnp.int32),
                       pltpu.VMEM((SC_CH, D), jnp.float32),
                       pltpu.SemaphoreType.DMA],
    )
    def k(table_hbm, idx_hbm, out_hbm, idx_v, rows_v, sem):
        wid = lax.axis_index("s") * 2 + lax.axis_index("c")
        base = wid * per

        def body(i, carry):
            b = base + i * SC_CH
            pltpu.sync_copy(idx_hbm.at[pl.ds(b, SC_CH)], idx_v)
            pltpu.async_copy(table_hbm.at[idx_v], rows_v, sem).wait()
            pltpu.sync_copy(rows_v, out_hbm.at[pl.ds(b, SC_CH)])
            return carry

        lax.fori_loop(0, nch, body, 0)

    return k(table, idx)


def kernel(coords, feats, W_in, b_in,
           g1_kw1, g1_kb1, g1_kw2, g1_kb2, g1_vw, g1_vb, g1_lng, g1_lnb,
           g1_sew1, g1_sew2,
           g2_kw1, g2_kb1, g2_kw2, g2_kb2, g2_vw, g2_vb, g2_lng, g2_lnb,
           g2_sew1, g2_sew2,
           W_lat, b_lat, p_kw1, p_kb1, p_kw2, p_kb2, p_lng, p_lnb):
    c2 = coords.reshape(N, 3)
    f2 = feats.reshape(N, 9)
    ct = c2.T

    dst = _knn(ct, c2)

    tbl1 = _tbl_in(f2, c2, W_in, b_in.reshape(1, -1),
                   g1_vw, g1_vb.reshape(1, -1))
    ga1 = _sc_gather(tbl1, dst.reshape(-1), PAD1, N * KG)
    o1, bs1 = _gno(ga1, tbl1,
                   g1_kw1, g1_kb1.reshape(1, -1), g1_kw2,
                   g1_kb2.reshape(1, -1))
    tbl2 = _scale(o1, tbl1, bs1.reshape(NBG, 64), g1_sew1, g1_sew2,
                  g1_lng.reshape(1, -1), g1_lnb.reshape(1, -1),
                  g2_vw, g2_vb.reshape(1, -1), PAD1)

    ga2 = _sc_gather(tbl2, dst.reshape(-1), PAD1, N * KG)
    o2, bs2 = _gno(ga2, tbl2,
                   g2_kw1, g2_kb1.reshape(1, -1), g2_kw2,
                   g2_kb2.reshape(1, -1))
    tbl3 = _scale(o2, tbl2, bs2.reshape(NBG, 64), g2_sew1, g2_sew2,
                  g2_lng.reshape(1, -1), g2_lnb.reshape(1, -1),
                  W_lat, b_lat.reshape(1, -1), PAD2)

    z = jnp.linspace(-1.0, 1.0, 32)
    gz, gy, gx = jnp.meshgrid(z, z, z, indexing='ij')
    grid_c = jnp.stack([gx, gy, gz], axis=-1).reshape(GM, 3)

    dist8, idx8 = _gknn(ct, grid_c)
    ga3 = _sc_gather(tbl3, idx8.reshape(-1), PAD2, GM * KP)
    out = _proj(ga3, dist8, grid_c,
                p_kw1, p_kb1.reshape(1, -1), p_kw2, p_kb2.reshape(1, -1),
                p_lng.reshape(1, -1), p_lnb.reshape(1, -1))

    return out.reshape(1, 32, 32, 32, 128).transpose(0, 4, 1, 2, 3)


# speedup vs baseline: 12.0091x; 1.0129x over previous
import functools

import numpy as np
import jax
import jax.numpy as jnp
from jax import lax
from jax.experimental import pallas as pl
from jax.experimental.pallas import tpu as pltpu
from jax.experimental.pallas import tpu_sc as plsc

N = 8192
KG = 12
KP = 8
GM = 32 * 32 * 32
PAD1 = 128
PAD2 = 256
BIGI = np.int32(2 ** 30)


def _dot(a, b):
    return lax.dot_general(
        a.astype(jnp.bfloat16), b.astype(jnp.bfloat16),
        (((1,), (0,)), ((), ())), preferred_element_type=jnp.float32)


def _gelu(x):
    return x * (lax.erf(x / np.sqrt(2).astype(np.float32)) + 1.0) / 2.0


def _sigmoid(x):
    return 1.0 / (1.0 + jnp.exp(-x))


def _ln(x, g, b, eps=1e-5):
    m = jnp.mean(x, axis=-1, keepdims=True)
    v = jnp.mean((x - m) * (x - m), axis=-1, keepdims=True)
    return (x - m) / jnp.sqrt(v + eps) * g + b


BRK = 256


IDXM = np.int32(8191)
IMAX = np.int32(2 ** 31 - 1)


def _pack(d2, cols):
    d2i = lax.bitcast_convert_type(d2, jnp.int32)
    return (d2i & ~IDXM) | cols


def _knn_body(ct_ref, a_ref, dst_ref):
    ct = ct_ref[...]
    a = a_ref[...]
    b2 = jnp.sum(ct * ct, axis=0, keepdims=True)
    a2 = jnp.sum(a * a, axis=1, keepdims=True)
    d2 = jnp.maximum(a2 + b2 - 2.0 * _dot(a, ct), 0.0)
    i = pl.program_id(0)
    rows = i * BRK + lax.broadcasted_iota(jnp.int32, (BRK, N), 0)
    cols = lax.broadcasted_iota(jnp.int32, (BRK, N), 1)
    d2 = jnp.where(cols == rows, jnp.inf, d2)
    outs = []
    for _ in range(KG):
        j = jnp.argmin(d2, axis=1, keepdims=True).astype(jnp.int32)
        outs.append(j)
        d2 = jnp.where(cols == j, jnp.inf, d2)
    dst_ref[...] = jnp.concatenate(outs, axis=1)


def _knn(ct, c2):
    return pl.pallas_call(
        _knn_body,
        grid=(N // BRK,),
        in_specs=[pl.BlockSpec((3, N), lambda i: (0, 0)),
                  pl.BlockSpec((BRK, 3), lambda i: (i, 0))],
        out_specs=pl.BlockSpec((BRK, KG), lambda i: (i, 0)),
        out_shape=jax.ShapeDtypeStruct((N, KG), jnp.int32),
    )(ct, c2)


BRG = 256


def _gknn_body(ct_ref, g_ref, dist_ref, idx_ref):
    ct = ct_ref[...]
    a = g_ref[...]
    b2 = jnp.sum(ct * ct, axis=0, keepdims=True)
    a2 = jnp.sum(a * a, axis=1, keepdims=True)
    d2 = jnp.maximum(a2 + b2 - 2.0 * _dot(a, ct), 0.0)
    cols = lax.broadcasted_iota(jnp.int32, (BRG, N), 1)
    dcols, icols = [], []
    for _ in range(KP):
        m = jnp.min(d2, axis=1, keepdims=True)
        j = jnp.argmin(d2, axis=1, keepdims=True).astype(jnp.int32)
        icols.append(j)
        dcols.append(m)
        d2 = jnp.where(cols == j, jnp.inf, d2)
    dist_ref[...] = jnp.sqrt(jnp.concatenate(dcols, axis=1))
    idx_ref[...] = jnp.concatenate(icols, axis=1)


def _gknn(ct, grid_c):
    return pl.pallas_call(
        _gknn_body,
        grid=(GM // BRG,),
        in_specs=[pl.BlockSpec((3, N), lambda i: (0, 0)),
                  pl.BlockSpec((BRG, 3), lambda i: (i, 0))],
        out_specs=[pl.BlockSpec((BRG, KP), lambda i: (i, 0)),
                   pl.BlockSpec((BRG, KP), lambda i: (i, 0))],
        out_shape=[jax.ShapeDtypeStruct((GM, KP), jnp.float32),
                   jax.ShapeDtypeStruct((GM, KP), jnp.int32)],
    )(ct, grid_c)


def _tbl_in_body(f_ref, c_ref, win_ref, bin_ref, vw_ref, vb_ref, tbl_ref):
    x = _dot(f_ref[...], win_ref[...]) + bin_ref[...]
    v = _dot(x, vw_ref[...]) + vb_ref[...]
    z = jnp.zeros((N, PAD1 - 67), jnp.float32)
    tbl_ref[...] = jnp.concatenate([v, c_ref[...], z], axis=1)


def _tbl_in(f2, c2, w_in, b_in, vw, vb):
    return pl.pallas_call(
        _tbl_in_body,
        out_shape=jax.ShapeDtypeStruct((N, PAD1), jnp.float32),
    )(f2, c2, w_in, b_in, vw, vb)


BRN = 512
NBN = N // BRN


BRN2 = 256
NBG = N // BRN2


def _gno_body(ga_ref, tbl_ref, kw1_ref, kb1_ref, kw2_ref, kb2_ref,
              out_ref, bs_ref):
    tbl = tbl_ref[...]
    c = tbl[:, 64:67]
    ga = ga_ref[...].reshape(BRN2, KG, PAD1)
    rel = (ga[:, :, 64:67] - c[:, None, :]).reshape(BRN2 * KG, 3)
    kap = _dot(_gelu(_dot(rel, kw1_ref[...]) + kb1_ref[...]),
               kw2_ref[...]) + kb2_ref[...]
    msg = kap.reshape(BRN2, KG, 64) * ga[:, :, 0:64]
    acc = jnp.sum(msg, axis=1)
    out_ref[...] = acc
    bs_ref[...] = jnp.sum(acc, axis=0)[None, None, :]


def _gno(ga, tbl, kw1, kb1, kw2, kb2):
    return pl.pallas_call(
        _gno_body,
        grid=(NBG,),
        in_specs=[pl.BlockSpec((BRN2 * KG, PAD1), lambda i: (i, 0)),
                  pl.BlockSpec((BRN2, PAD1), lambda i: (i, 0)),
                  pl.BlockSpec((3, 32), lambda i: (0, 0)),
                  pl.BlockSpec((1, 32), lambda i: (0, 0)),
                  pl.BlockSpec((32, 64), lambda i: (0, 0)),
                  pl.BlockSpec((1, 64), lambda i: (0, 0))],
        out_specs=[pl.BlockSpec((BRN2, 64), lambda i: (i, 0)),
                   pl.BlockSpec((1, 1, 64), lambda i: (i, 0, 0))],
        out_shape=[jax.ShapeDtypeStruct((N, 64), jnp.float32),
                   jax.ShapeDtypeStruct((NBG, 1, 64), jnp.float32)],
    )(ga, tbl, kw1, kb1, kw2, kb2)


def _scale_body(out_ref, tbl_ref, bs_ref, sew1_ref, sew2_ref, lng_ref,
                lnb_ref, pw_ref, pb_ref, ntbl_ref, *, pad):
    ym = jnp.sum(bs_ref[...], axis=0, keepdims=True) / np.float32(N)
    sc = _sigmoid(_dot(_gelu(_dot(ym, sew1_ref[...])), sew2_ref[...]))
    tbl = tbl_ref[...]
    v = tbl[:, 0:64]
    c = tbl[:, 64:67]
    x2 = _ln(_gelu(out_ref[...] * sc + v), lng_ref[...], lnb_ref[...])
    nv = _dot(x2, pw_ref[...]) + pb_ref[...]
    z = jnp.zeros((BRN, pad - nv.shape[1] - 3), jnp.float32)
    ntbl_ref[...] = jnp.concatenate([nv, c, z], axis=1)


def _scale(out, tbl, bs, sew1, sew2, lng, lnb, pw, pb, pad):
    pd = pw.shape[1]
    return pl.pallas_call(
        functools.partial(_scale_body, pad=pad),
        grid=(NBN,),
        in_specs=[pl.BlockSpec((BRN, 64), lambda i: (i, 0)),
                  pl.BlockSpec((BRN, PAD1), lambda i: (i, 0)),
                  pl.BlockSpec((NBG, 64), lambda i: (0, 0)),
                  pl.BlockSpec((64, 16), lambda i: (0, 0)),
                  pl.BlockSpec((16, 64), lambda i: (0, 0)),
                  pl.BlockSpec((1, 64), lambda i: (0, 0)),
                  pl.BlockSpec((1, 64), lambda i: (0, 0)),
                  pl.BlockSpec((64, pd), lambda i: (0, 0)),
                  pl.BlockSpec((1, pd), lambda i: (0, 0))],
        out_specs=pl.BlockSpec((BRN, pad), lambda i: (i, 0)),
        out_shape=jax.ShapeDtypeStruct((N, pad), jnp.float32),
    )(out, tbl, bs, sew1, sew2, lng, lnb, pw, pb)


BRP = 256


def _proj_body(ga_ref, dist_ref, gc_ref, kw1_ref, kb1_ref, kw2_ref,
               kb2_ref, lng_ref, lnb_ref, out_ref):
    w = 1.0 / (dist_ref[...] + 1e-6)
    w = w / jnp.sum(w, axis=1, keepdims=True)
    gc = gc_ref[...]
    ga = ga_ref[...].reshape(BRP, KP, PAD2)
    rel = (gc[:, None, :] - ga[:, :, 128:131]).reshape(BRP * KP, 3)
    kap = _dot(_gelu(_dot(rel, kw1_ref[...]) + kb1_ref[...]),
               kw2_ref[...]) + kb2_ref[...]
    msg = kap.reshape(BRP, KP, 128) * ga[:, :, 0:128] * w[:, :, None]
    acc = jnp.sum(msg, axis=1)
    out_ref[...] = _ln(acc, lng_ref[...], lnb_ref[...])


def _proj(ga, dist8, grid_c, kw1, kb1, kw2, kb2, lng, lnb):
    return pl.pallas_call(
        _proj_body,
        grid=(GM // BRP,),
        in_specs=[pl.BlockSpec((BRP * KP, PAD2), lambda i: (i, 0)),
                  pl.BlockSpec((BRP, KP), lambda i: (i, 0)),
                  pl.BlockSpec((BRP, 3), lambda i: (i, 0)),
                  pl.BlockSpec((3, 128), lambda i: (0, 0)),
                  pl.BlockSpec((1, 128), lambda i: (0, 0)),
                  pl.BlockSpec((128, 128), lambda i: (0, 0)),
                  pl.BlockSpec((1, 128), lambda i: (0, 0)),
                  pl.BlockSpec((1, 128), lambda i: (0, 0)),
                  pl.BlockSpec((1, 128), lambda i: (0, 0))],
        out_specs=pl.BlockSpec((BRP, 128), lambda i: (i, 0)),
        out_shape=jax.ShapeDtypeStruct((GM, 128), jnp.float32),
    )(ga, dist8, grid_c, kw1, kb1, kw2, kb2, lng, lnb)


NW = 32


def _sc_gather(table, idx, D, B):
    SC_CH = 512 if D <= 128 else 256
    per = B // NW
    nch = per // SC_CH
    mesh = plsc.VectorSubcoreMesh(core_axis_name="c", subcore_axis_name="s")

    @functools.partial(
        pl.kernel, mesh=mesh,
        out_type=jax.ShapeDtypeStruct((B, D), jnp.float32),
        scratch_types=[pltpu.VMEM((SC_CH,), jnp.int32),
                       pltpu.VMEM((SC_CH, D), jnp.float32),
                       pltpu.SemaphoreType.DMA],
    )
    def k(table_hbm, idx_hbm, out_hbm, idx_v, rows_v, sem):
        wid = lax.axis_index("s") * 2 + lax.axis_index("c")
        base = wid * per

        def body(i, carry):
            b = base + i * SC_CH
            pltpu.sync_copy(idx_hbm.at[pl.ds(b, SC_CH)], idx_v)
            pltpu.async_copy(table_hbm.at[idx_v], rows_v, sem).wait()
            pltpu.sync_copy(rows_v, out_hbm.at[pl.ds(b, SC_CH)])
            return carry

        lax.fori_loop(0, nch, body, 0)

    return k(table, idx)


def kernel(coords, feats, W_in, b_in,
           g1_kw1, g1_kb1, g1_kw2, g1_kb2, g1_vw, g1_vb, g1_lng, g1_lnb,
           g1_sew1, g1_sew2,
           g2_kw1, g2_kb1, g2_kw2, g2_kb2, g2_vw, g2_vb, g2_lng, g2_lnb,
           g2_sew1, g2_sew2,
           W_lat, b_lat, p_kw1, p_kb1, p_kw2, p_kb2, p_lng, p_lnb):
    c2 = coords.reshape(N, 3)
    f2 = feats.reshape(N, 9)
    ct = c2.T

    dst = _knn(ct, c2)

    tbl1 = _tbl_in(f2, c2, W_in, b_in.reshape(1, -1),
                   g1_vw, g1_vb.reshape(1, -1))
    ga1 = _sc_gather(tbl1, dst.reshape(-1), PAD1, N * KG)
    o1, bs1 = _gno(ga1, tbl1,
                   g1_kw1, g1_kb1.reshape(1, -1), g1_kw2,
                   g1_kb2.reshape(1, -1))
    tbl2 = _scale(o1, tbl1, bs1.reshape(NBG, 64), g1_sew1, g1_sew2,
                  g1_lng.reshape(1, -1), g1_lnb.reshape(1, -1),
                  g2_vw, g2_vb.reshape(1, -1), PAD1)

    ga2 = _sc_gather(tbl2, dst.reshape(-1), PAD1, N * KG)
    o2, bs2 = _gno(ga2, tbl2,
                   g2_kw1, g2_kb1.reshape(1, -1), g2_kw2,
                   g2_kb2.reshape(1, -1))
    tbl3 = _scale(o2, tbl2, bs2.reshape(NBG, 64), g2_sew1, g2_sew2,
                  g2_lng.reshape(1, -1), g2_lnb.reshape(1, -1),
                  W_lat, b_lat.reshape(1, -1), PAD2)

    z = jnp.linspace(-1.0, 1.0, 32)
    gz, gy, gx = jnp.meshgrid(z, z, z, indexing='ij')
    grid_c = jnp.stack([gx, gy, gz], axis=-1).reshape(GM, 3)

    dist8, idx8 = _gknn(ct, grid_c)
    ga3 = _sc_gather(tbl3, idx8.reshape(-1), PAD2, GM * KP)
    out = _proj(ga3, dist8, grid_c,
                p_kw1, p_kb1.reshape(1, -1), p_kw2, p_kb2.reshape(1, -1),
                p_lng.reshape(1, -1), p_lnb.reshape(1, -1))

    return out.reshape(1, 32, 32, 32, 128).transpose(0, 4, 1, 2, 3)
